# baseline (speedup 1.0000x reference)
import functools

import jax
import jax.numpy as jnp
from jax import lax
from jax.experimental import pallas as pl

_S = 0.7071067811865476
_W2 = 112


def _fourier_body(x_ref, add_ref, sub_ref, *, rows, w2):
    c = add_ref.shape[-1]
    ngroups = w2 // 8
    c_chunks = []
    c0 = 0
    while c0 < c:
        cw = min(128, c - c0)
        c_chunks.append((c0, cw))
        c0 += cw
    for h in range(rows):
        x = x_ref[0, :, h, :]
        for c0, cw in c_chunks:
            xt = jnp.transpose(x[c0:c0 + cw, :])
            l = xt[:w2, :]
            b = xt[w2:, :]
            rev8 = 7 - lax.broadcasted_iota(jnp.int32, (8, cw), 0)
            for g in range(ngroups):
                src = b[w2 - 8 * (g + 1): w2 - 8 * g, :]
                r = jnp.take_along_axis(src, rev8, axis=0)
                lg = l[8 * g: 8 * (g + 1), :]
                base = h * w2 + 8 * g
                add_ref[0, base: base + 8, c0:c0 + cw] = _S * (lg + r)
                sub_ref[0, base: base + 8, c0:c0 + cw] = _S * (lg - r)


def kernel(im, left_idx, right_idx):
    del left_idx, right_idx
    B, C, H, W = im.shape
    w2 = W // 2
    rows = 8
    out_sd = jax.ShapeDtypeStruct((B, H * w2, C), im.dtype)
    add_o, sub_o = pl.pallas_call(
        functools.partial(_fourier_body, rows=rows, w2=w2),
        grid=(B, H // rows),
        in_specs=[pl.BlockSpec((1, C, rows, W), lambda b, i: (b, 0, i, 0))],
        out_specs=[
            pl.BlockSpec((1, rows * w2, C), lambda b, i: (b, i, 0)),
            pl.BlockSpec((1, rows * w2, C), lambda b, i: (b, i, 0)),
        ],
        out_shape=[out_sd, out_sd],
    )(im)
    return (add_o, sub_o)


# speedup vs baseline: 1.1212x; 1.1212x over previous
import functools

import jax
import jax.numpy as jnp
from jax import lax
from jax.experimental import pallas as pl

_S = 0.7071067811865476
_W2 = 112


def _fourier_body(x_ref, add_ref, sub_ref, *, rows, w2):
    c = add_ref.shape[-1]
    ngroups = w2 // 8
    c_chunks = []
    c0 = 0
    while c0 < c:
        cw = min(128, c - c0)
        c_chunks.append((c0, cw))
        c0 += cw
    for h in range(rows):
        x = x_ref[0, :, h, :]
        for c0, cw in c_chunks:
            xt = jnp.transpose(x[c0:c0 + cw, :])
            l = xt[:w2, :]
            b = xt[w2:, :]
            rev8 = 7 - lax.broadcasted_iota(jnp.int32, (8, cw), 0)
            for g in range(ngroups):
                src = b[w2 - 8 * (g + 1): w2 - 8 * g, :]
                r = jnp.take_along_axis(src, rev8, axis=0)
                lg = l[8 * g: 8 * (g + 1), :]
                base = h * w2 + 8 * g
                add_ref[0, base: base + 8, c0:c0 + cw] = _S * (lg + r)
                sub_ref[0, base: base + 8, c0:c0 + cw] = _S * (lg - r)


def kernel(im, left_idx, right_idx):
    del left_idx, right_idx
    B, C, H, W = im.shape
    w2 = W // 2
    rows = 32
    out_sd = jax.ShapeDtypeStruct((B, H * w2, C), im.dtype)
    add_o, sub_o = pl.pallas_call(
        functools.partial(_fourier_body, rows=rows, w2=w2),
        grid=(B, H // rows),
        in_specs=[pl.BlockSpec((1, C, rows, W), lambda b, i: (b, 0, i, 0))],
        out_specs=[
            pl.BlockSpec((1, rows * w2, C), lambda b, i: (b, i, 0)),
            pl.BlockSpec((1, rows * w2, C), lambda b, i: (b, i, 0)),
        ],
        out_shape=[out_sd, out_sd],
    )(im)
    return (add_o, sub_o)


# speedup vs baseline: 1.1553x; 1.0304x over previous
import functools

import jax
import jax.numpy as jnp
from jax import lax
from jax.experimental import pallas as pl
from jax.experimental.pallas import tpu as pltpu

_S = 0.7071067811865476

_ROWS = 16
_NBUF = 3
_KC = 4
_KL = 2


def _compute_tile(x_buf, a_buf, s_buf, slot, rows, w2, c):
    ngroups = w2 // 8
    c_chunks = []
    c0 = 0
    while c0 < c:
        cw = min(128, c - c0)
        c_chunks.append((c0, cw))
        c0 += cw
    for h in range(rows):
        x = x_buf[slot, :, h, :]
        for c0, cw in c_chunks:
            xt = jnp.transpose(x[c0:c0 + cw, :])
            l = xt[:w2, :]
            b = xt[w2:, :]
            rev8 = 7 - lax.broadcasted_iota(jnp.int32, (8, cw), 0)
            for g in range(ngroups):
                src = b[w2 - 8 * (g + 1): w2 - 8 * g, :]
                r = jnp.take_along_axis(src, rev8, axis=0)
                lg = l[8 * g: 8 * (g + 1), :]
                base = h * w2 + 8 * g
                a_buf[slot, base: base + 8, c0:c0 + cw] = _S * (lg + r)
                s_buf[slot, base: base + 8, c0:c0 + cw] = _S * (lg - r)


def _pipeline_body(im_ref, add_ref, sub_ref,
                   x_buf, a_buf, s_buf, in_sems, out_sems,
                   *, B, C, H, W, rows):
    w2 = W // 2
    n_i = H // rows
    nsteps = B * n_i
    cc = C // _KC
    lblk = rows * w2
    lc = lblk // _KL

    def in_copies(step, slot):
        b = step // n_i
        i = step % n_i
        cps = []
        for k in range(_KC):
            cps.append(pltpu.make_async_copy(
                im_ref.at[b, pl.ds(k * cc, cc), pl.ds(i * rows, rows), :],
                x_buf.at[slot, pl.ds(k * cc, cc)],
                in_sems.at[slot, k]))
        return cps

    def out_copies(step, slot):
        b = step // n_i
        i = step % n_i
        cps = []
        for k in range(_KL):
            cps.append(pltpu.make_async_copy(
                a_buf.at[slot, pl.ds(k * lc, lc)],
                add_ref.at[b, pl.ds(i * lblk + k * lc, lc)],
                out_sems.at[slot, k]))
            cps.append(pltpu.make_async_copy(
                s_buf.at[slot, pl.ds(k * lc, lc)],
                sub_ref.at[b, pl.ds(i * lblk + k * lc, lc)],
                out_sems.at[slot, _KL + k]))
        return cps

    for s in range(min(nsteps, _NBUF - 1)):
        for cp in in_copies(s, s % _NBUF):
            cp.start()

    def loop(s, carry):
        slot = lax.rem(s, _NBUF)

        @pl.when(s + _NBUF - 1 < nsteps)
        def _():
            for cp in in_copies(s + _NBUF - 1, lax.rem(s + _NBUF - 1, _NBUF)):
                cp.start()

        for cp in in_copies(s, slot):
            cp.wait()

        @pl.when(s >= _NBUF)
        def _():
            for cp in out_copies(s - _NBUF, slot):
                cp.wait()

        _compute_tile(x_buf, a_buf, s_buf, slot, rows, w2, C)

        for cp in out_copies(s, slot):
            cp.start()
        return carry

    lax.fori_loop(0, nsteps, loop, 0)

    for s in range(max(0, nsteps - _NBUF), nsteps):
        for cp in out_copies(s, s % _NBUF):
            cp.wait()


def kernel(im, left_idx, right_idx):
    del left_idx, right_idx
    B, C, H, W = im.shape
    w2 = W // 2
    out_sd = jax.ShapeDtypeStruct((B, H * w2, C), im.dtype)
    add_o, sub_o = pl.pallas_call(
        functools.partial(_pipeline_body, B=B, C=C, H=H, W=W, rows=_ROWS),
        in_specs=[pl.BlockSpec(memory_space=pl.ANY)],
        out_specs=[pl.BlockSpec(memory_space=pl.ANY),
                   pl.BlockSpec(memory_space=pl.ANY)],
        out_shape=[out_sd, out_sd],
        scratch_shapes=[
            pltpu.VMEM((_NBUF, C, _ROWS, W), im.dtype),
            pltpu.VMEM((_NBUF, _ROWS * w2, C), im.dtype),
            pltpu.VMEM((_NBUF, _ROWS * w2, C), im.dtype),
            pltpu.SemaphoreType.DMA((_NBUF, _KC)),
            pltpu.SemaphoreType.DMA((_NBUF, 2 * _KL)),
        ],
    )(im)
    return (add_o, sub_o)


# speedup vs baseline: 1.1591x; 1.0034x over previous
import functools

import jax
import jax.numpy as jnp
from jax import lax
from jax.experimental import pallas as pl
from jax.experimental.pallas import tpu as pltpu

_S = 0.7071067811865476

_ROWS = 16
_NBUF = 4
_KC = 4
_KL = 2


def _compute_tile(x_buf, a_buf, s_buf, slot, rows, w2, c):
    ngroups = w2 // 8
    c_chunks = []
    c0 = 0
    while c0 < c:
        cw = min(128, c - c0)
        c_chunks.append((c0, cw))
        c0 += cw
    for h in range(rows):
        x = x_buf[slot, :, h, :]
        for c0, cw in c_chunks:
            xt = jnp.transpose(x[c0:c0 + cw, :])
            l = xt[:w2, :]
            b = xt[w2:, :]
            rev8 = 7 - lax.broadcasted_iota(jnp.int32, (8, cw), 0)
            for g in range(ngroups):
                src = b[w2 - 8 * (g + 1): w2 - 8 * g, :]
                r = jnp.take_along_axis(src, rev8, axis=0)
                lg = l[8 * g: 8 * (g + 1), :]
                base = h * w2 + 8 * g
                a_buf[slot, base: base + 8, c0:c0 + cw] = _S * (lg + r)
                s_buf[slot, base: base + 8, c0:c0 + cw] = _S * (lg - r)


def _pipeline_body(im_ref, add_ref, sub_ref,
                   x_buf, a_buf, s_buf, in_sems, out_sems,
                   *, B, C, H, W, rows):
    w2 = W // 2
    n_i = H // rows
    nsteps = B * n_i
    cc = C // _KC
    lblk = rows * w2
    lc = lblk // _KL

    def in_copies(step, slot):
        b = step // n_i
        i = step % n_i
        cps = []
        for k in range(_KC):
            cps.append(pltpu.make_async_copy(
                im_ref.at[b, pl.ds(k * cc, cc), pl.ds(i * rows, rows), :],
                x_buf.at[slot, pl.ds(k * cc, cc)],
                in_sems.at[slot, k]))
        return cps

    def out_copies(step, slot):
        b = step // n_i
        i = step % n_i
        cps = []
        for k in range(_KL):
            cps.append(pltpu.make_async_copy(
                a_buf.at[slot, pl.ds(k * lc, lc)],
                add_ref.at[b, pl.ds(i * lblk + k * lc, lc)],
                out_sems.at[slot, k]))
            cps.append(pltpu.make_async_copy(
                s_buf.at[slot, pl.ds(k * lc, lc)],
                sub_ref.at[b, pl.ds(i * lblk + k * lc, lc)],
                out_sems.at[slot, _KL + k]))
        return cps

    for s in range(min(nsteps, _NBUF - 1)):
        for cp in in_copies(s, s % _NBUF):
            cp.start()

    def loop(s, carry):
        slot = lax.rem(s, _NBUF)

        @pl.when(s + _NBUF - 1 < nsteps)
        def _():
            for cp in in_copies(s + _NBUF - 1, lax.rem(s + _NBUF - 1, _NBUF)):
                cp.start()

        for cp in in_copies(s, slot):
            cp.wait()

        @pl.when(s >= _NBUF)
        def _():
            for cp in out_copies(s - _NBUF, slot):
                cp.wait()

        _compute_tile(x_buf, a_buf, s_buf, slot, rows, w2, C)

        for cp in out_copies(s, slot):
            cp.start()
        return carry

    lax.fori_loop(0, nsteps, loop, 0)

    for s in range(max(0, nsteps - _NBUF), nsteps):
        for cp in out_copies(s, s % _NBUF):
            cp.wait()


def kernel(im, left_idx, right_idx):
    del left_idx, right_idx
    B, C, H, W = im.shape
    w2 = W // 2
    out_sd = jax.ShapeDtypeStruct((B, H * w2, C), im.dtype)
    add_o, sub_o = pl.pallas_call(
        functools.partial(_pipeline_body, B=B, C=C, H=H, W=W, rows=_ROWS),
        in_specs=[pl.BlockSpec(memory_space=pl.ANY)],
        out_specs=[pl.BlockSpec(memory_space=pl.ANY),
                   pl.BlockSpec(memory_space=pl.ANY)],
        out_shape=[out_sd, out_sd],
        scratch_shapes=[
            pltpu.VMEM((_NBUF, C, _ROWS, W), im.dtype),
            pltpu.VMEM((_NBUF, _ROWS * w2, C), im.dtype),
            pltpu.VMEM((_NBUF, _ROWS * w2, C), im.dtype),
            pltpu.SemaphoreType.DMA((_NBUF, _KC)),
            pltpu.SemaphoreType.DMA((_NBUF, 2 * _KL)),
        ],
    )(im)
    return (add_o, sub_o)
